# Initial kernel scaffold; baseline (speedup 1.0000x reference)
import jax
import jax.numpy as jnp
from jax.experimental import pallas as pl


def kernel(x, adj, W1, b1, W2, b2):
    raise NotImplementedError("write your pallas kernel here")


# speedup vs baseline: 10.6677x; 10.6677x over previous
import functools

import jax
import jax.numpy as jnp
from jax import lax
from jax.experimental import pallas as pl
from jax.experimental.pallas import tpu as pltpu
from jax.experimental.pallas import tpu_sc as plsc

N = 10000
D = 128
E = 320000
CH = 128
NC = 2
NS = 16
NW = NC * NS
NCH = -(-E // (NW * CH))
E_PAD = NW * CH * NCH
NPAD = 10112
RPT = NPAD // NS

_mesh = plsc.VectorSubcoreMesh(core_axis_name="c", subcore_axis_name="s")


@functools.partial(
    pl.kernel,
    out_type=jax.ShapeDtypeStruct((NC, NPAD, D), jnp.float32),
    mesh=_mesh,
    scratch_types=[
        pltpu.VMEM((CH,), jnp.int32),
        pltpu.VMEM((CH, D), jnp.float32),
        pltpu.VMEM_SHARED((NPAD, D), jnp.float32),
        pltpu.SemaphoreType.DMA,
    ],
)
def _deg_kernel(dst_hbm, ones_hbm, zeros_hbm, out_hbm, idx_v, ones_v, acc, sem):
    c = lax.axis_index("c")
    s = lax.axis_index("s")
    wid = s * NC + c
    pltpu.sync_copy(zeros_hbm.at[pl.ds(s * RPT, RPT)], acc.at[pl.ds(s * RPT, RPT)])
    pltpu.sync_copy(ones_hbm, ones_v)
    plsc.subcore_barrier()
    base = wid * (NCH * CH)

    def body(i, carry):
        pltpu.sync_copy(dst_hbm.at[pl.ds(base + i * CH, CH)], idx_v)
        pltpu.sync_copy(ones_v, acc.at[idx_v], add=True)
        return carry

    lax.fori_loop(0, NCH, body, 0)
    plsc.subcore_barrier()
    pltpu.sync_copy(acc.at[pl.ds(s * RPT, RPT)], out_hbm.at[c, pl.ds(s * RPT, RPT)])


@functools.partial(
    pl.kernel,
    out_type=jax.ShapeDtypeStruct((NC, NPAD, D), jnp.float32),
    mesh=_mesh,
    scratch_types=[
        pltpu.VMEM((CH,), jnp.int32),
        pltpu.VMEM((CH,), jnp.int32),
        pltpu.VMEM((CH, D), jnp.float32),
        pltpu.VMEM_SHARED((NPAD, D), jnp.float32),
        pltpu.SemaphoreType.DMA,
    ],
)
def _msg_kernel(src_hbm, dst_hbm, h_hbm, zeros_hbm, out_hbm,
                sidx, didx, rows, acc, sem):
    c = lax.axis_index("c")
    s = lax.axis_index("s")
    wid = s * NC + c
    pltpu.sync_copy(zeros_hbm.at[pl.ds(s * RPT, RPT)], acc.at[pl.ds(s * RPT, RPT)])
    plsc.subcore_barrier()
    base = wid * (NCH * CH)

    def body(i, carry):
        off = base + i * CH
        pltpu.sync_copy(src_hbm.at[pl.ds(off, CH)], sidx)
        pltpu.sync_copy(dst_hbm.at[pl.ds(off, CH)], didx)
        pltpu.async_copy(h_hbm.at[sidx], rows, sem).wait()
        pltpu.sync_copy(rows, acc.at[didx], add=True)
        return carry

    lax.fori_loop(0, NCH, body, 0)
    plsc.subcore_barrier()
    pltpu.sync_copy(acc.at[pl.ds(s * RPT, RPT)], out_hbm.at[c, pl.ds(s * RPT, RPT)])


def _tca_body(parts_ref, x_ref, w1_ref, dis_ref, h1p_ref):
    deg = jnp.sum(parts_ref[0, :N, :] + parts_ref[1, :N, :], axis=1,
                  keepdims=True) * (1.0 / D) + 1.0
    dis = lax.rsqrt(deg)
    dis_ref[...] = dis
    h1p_ref[...] = dis * jnp.dot(x_ref[...], w1_ref[...],
                                 preferred_element_type=jnp.float32)


_tca = pl.pallas_call(
    _tca_body,
    out_shape=(jax.ShapeDtypeStruct((N, 1), jnp.float32),
               jax.ShapeDtypeStruct((N, D), jnp.float32)),
)


def _tcb_body(acc_ref, dis_ref, h1p_ref, b1_ref, w2_ref, h2p_ref):
    dis = dis_ref[...]
    z = dis * (acc_ref[0, :N, :] + acc_ref[1, :N, :] + h1p_ref[...]) + b1_ref[...]
    z = jnp.maximum(z, 0.0)
    h2p_ref[...] = dis * jnp.dot(z, w2_ref[...],
                                 preferred_element_type=jnp.float32)


_tcb = pl.pallas_call(
    _tcb_body,
    out_shape=jax.ShapeDtypeStruct((N, D), jnp.float32),
)


def _tcc_body(acc_ref, dis_ref, h2p_ref, b2_ref, out_ref):
    out_ref[...] = (dis_ref[...]
                    * (acc_ref[0, :N, :] + acc_ref[1, :N, :] + h2p_ref[...])
                    + b2_ref[...])


_tcc = pl.pallas_call(
    _tcc_body,
    out_shape=jax.ShapeDtypeStruct((N, D), jnp.float32),
)


def kernel(x, adj, W1, b1, W2, b2):
    src = adj[0].astype(jnp.int32)
    dst = adj[1].astype(jnp.int32)
    pad = E_PAD - E
    src_p = jnp.concatenate([src, jnp.zeros((pad,), jnp.int32)])
    dst_p = jnp.concatenate([dst, jnp.full((pad,), NPAD - 1, jnp.int32)])
    onesD = jnp.ones((CH, D), jnp.float32)
    zerosD = jnp.zeros((NPAD, D), jnp.float32)

    deg_parts = _deg_kernel(dst_p, onesD, zerosD)
    dis, h1p = _tca(deg_parts, x, W1)
    acc1 = _msg_kernel(src_p, dst_p, h1p, zerosD)
    h2p = _tcb(acc1, dis, h1p, b1.reshape(1, D), W2)
    acc2 = _msg_kernel(src_p, dst_p, h2p, zerosD)
    out = _tcc(acc2, dis, h2p, b2.reshape(1, D))
    return out
